# baseline (speedup 1.0000x reference)
import functools

import jax
import jax.numpy as jnp
from jax import lax
from jax.experimental import pallas as pl
from jax.experimental.pallas import tpu as pltpu
from jax.experimental.pallas import tpu_sc as plsc

K = 300
KPAD = 320
CAND = 384
CHUNK = 128
B, N, NCLS, DMEM = 16, 8192, 80, 256
INT_MIN32 = -2147483648


def _rowmax_body(cls_ref, out_ref, pad_ref):
    x = cls_ref[0]
    out_ref[0, 0, :] = jnp.max(x, axis=-1)
    pad_ref[0] = jnp.concatenate(
        [x, jnp.zeros((N // 4, 128 - NCLS), jnp.float32)], axis=-1)


def _row_max(cls):
    return pl.pallas_call(
        _rowmax_body,
        grid=(B * 4,),
        in_specs=[pl.BlockSpec((1, N // 4, NCLS), lambda i: (i // 4, i % 4, 0))],
        out_specs=[pl.BlockSpec((1, 1, N // 4), lambda i: (i, 0, 0)),
                   pl.BlockSpec((1, N // 4, 128), lambda i: (i // 4, i % 4, 0))],
        out_shape=[jax.ShapeDtypeStruct((B * 4, 1, N // 4), jnp.float32),
                   jax.ShapeDtypeStruct((B, N, 128), jnp.float32)],
    )(cls)


def _topk_body(cm_ref, coordt_ref, idx_ref, refp_ref, bbox_ref):
    b = pl.program_id(0)
    x = cm_ref[0]
    kb = lax.bitcast_convert_type(x, jnp.int32)
    m = jnp.where(kb < 0, kb ^ jnp.int32(0x7FFFFFFF), kb)
    m2 = m.reshape(64, 128)

    tb = jnp.int32(0)
    for bit in range(31, -1, -1):
        bv = jnp.int32(INT_MIN32) if bit == 31 else jnp.int32(1 << bit)
        cand = tb | bv
        ts = cand ^ jnp.int32(INT_MIN32)
        cnt = jnp.sum((m2 >= ts).astype(jnp.int32))
        tb = jnp.where(cnt >= K, cand, tb)
    ts = tb ^ jnp.int32(INT_MIN32)
    flag2 = m2 >= ts
    fl2 = flag2.astype(jnp.float32)
    count_ge = jnp.sum(fl2)

    lt128 = (lax.broadcasted_iota(jnp.int32, (128, 128), 0)
             < lax.broadcasted_iota(jnp.int32, (128, 128), 1)).astype(jnp.float32)
    ex = jnp.dot(fl2, lt128, preferred_element_type=jnp.float32)
    rows = jnp.sum(fl2, axis=1, keepdims=True)
    ltt64 = (lax.broadcasted_iota(jnp.int32, (64, 64), 1)
             < lax.broadcasted_iota(jnp.int32, (64, 64), 0)).astype(jnp.float32)
    rowex = jnp.dot(ltt64, rows, preferred_element_type=jnp.float32)
    pos2 = rowex + ex
    posm = jnp.where(flag2, pos2, -1.0).reshape(1, N)

    gidxf = (lax.broadcasted_iota(jnp.int32, (1, N), 1)
             + b * N).astype(jnp.float32)
    vt = jnp.concatenate([x, gidxf, coordt_ref[0]], axis=0)

    jio = lax.broadcasted_iota(jnp.int32, (CAND, N // 8), 0).astype(jnp.float32)
    acc = jnp.zeros((CAND, 6), jnp.float32)
    for c in range(8):
        sl = slice(c * (N // 8), (c + 1) * (N // 8))
        p = (jio == posm[:, sl]).astype(jnp.float32)
        acc = acc + lax.dot_general(p, vt[:, sl], (((1,), (1,)), ((), ())),
                                    precision=lax.Precision.HIGHEST,
                                    preferred_element_type=jnp.float32)

    slotc = lax.broadcasted_iota(jnp.int32, (CAND, 1), 0).astype(jnp.float32)
    valid = slotc < count_ge
    cval = jnp.where(valid, acc[:, 0:1], -jnp.inf)
    cidx = jnp.where(valid, acc[:, 1:2], jnp.float32(2 * B * N))

    eye = (lax.broadcasted_iota(jnp.int32, (CAND, CAND), 0)
           == lax.broadcasted_iota(jnp.int32, (CAND, CAND), 1)).astype(jnp.float32)
    avr = lax.dot_general(acc[:, 0:1], eye, (((0,), (0,)), ((), ())),
                          precision=lax.Precision.HIGHEST,
                          preferred_element_type=jnp.float32)
    air = lax.dot_general(acc[:, 1:2], eye, (((0,), (0,)), ((), ())),
                          precision=lax.Precision.HIGHEST,
                          preferred_element_type=jnp.float32)
    validr = (lax.broadcasted_iota(jnp.int32, (1, CAND), 1)
              .astype(jnp.float32) < count_ge)
    cvalr = jnp.where(validr, avr, -jnp.inf)
    cidxr = jnp.where(validr, air, jnp.float32(2 * B * N))

    gt = (cvalr > cval) | ((cvalr == cval) & (cidxr < cidx))
    cnt2 = jnp.sum((gt & validr).astype(jnp.float32), axis=1, keepdims=True)
    rank = jnp.where(valid, cnt2, jnp.float32(CAND))

    oh = (rank == lax.broadcasted_iota(jnp.int32, (CAND, KPAD), 1)
          .astype(jnp.float32)).astype(jnp.float32)
    v2 = jnp.concatenate([cidx, acc[:, 2:6]], axis=1)
    out5 = lax.dot_general(v2, oh, (((0,), (0,)), ((), ())),
                           precision=lax.Precision.HIGHEST,
                           preferred_element_type=jnp.float32)
    idx_ref[0] = out5[0:1, :].astype(jnp.int32)
    rp = out5[1:5, :]
    refp_ref[0] = rp
    bbox_ref[0] = jax.nn.sigmoid(rp)


def _topk(cm16, coordt):
    return pl.pallas_call(
        _topk_body,
        grid=(B,),
        in_specs=[pl.BlockSpec((1, 1, N), lambda i: (i, 0, 0)),
                  pl.BlockSpec((1, 4, N), lambda i: (i, 0, 0))],
        out_specs=[pl.BlockSpec((1, 1, KPAD), lambda i: (i, 0, 0)),
                   pl.BlockSpec((1, 4, KPAD), lambda i: (i, 0, 0)),
                   pl.BlockSpec((1, 4, KPAD), lambda i: (i, 0, 0))],
        out_shape=[jax.ShapeDtypeStruct((B, 1, KPAD), jnp.int32),
                   jax.ShapeDtypeStruct((B, 4, KPAD), jnp.float32),
                   jax.ShapeDtypeStruct((B, 4, KPAD), jnp.float32)],
    )(cm16, coordt)


def _sc_gather(flat_idx, cls3, mem2):
    info = plsc.get_sparse_core_info()
    nw = info.num_cores * info.num_subcores
    lanes = info.num_lanes
    tot = B * KPAD
    bpw = tot // nw
    nch = -(-bpw // 128)
    cw = bpw // nch
    mesh = plsc.VectorSubcoreMesh(core_axis_name="c", subcore_axis_name="s")

    @functools.partial(
        pl.kernel, mesh=mesh,
        out_type=[jax.ShapeDtypeStruct((tot, 128), jnp.float32),
                  jax.ShapeDtypeStruct((tot, DMEM), jnp.float32)],
        scratch_types=[pltpu.VMEM((bpw,), jnp.int32),
                       pltpu.VMEM((bpw, 128), jnp.float32),
                       pltpu.VMEM((bpw, DMEM), jnp.float32),
                       pltpu.SemaphoreType.DMA,
                       pltpu.SemaphoreType.DMA],
    )
    def gk(idx_hbm, cls_hbm, mem_hbm, ocls_hbm, omem_hbm,
           idx_v, crows, mrows, sem_c, sem_m):
        wid = lax.axis_index("s") * info.num_cores + lax.axis_index("c")
        base = wid * bpw
        pltpu.sync_copy(idx_hbm.at[pl.ds(base, bpw)], idx_v)
        cps = []
        for c in range(nch):
            sl = pl.ds(c * cw, cw)
            cps.append(pltpu.async_copy(cls_hbm.at[idx_v.at[sl]], crows.at[sl], sem_c))
            cps.append(pltpu.async_copy(mem_hbm.at[idx_v.at[sl]], mrows.at[sl], sem_m))
        for cp in cps:
            cp.wait()
        pltpu.sync_copy(crows, ocls_hbm.at[pl.ds(base, bpw)])
        pltpu.sync_copy(mrows, omem_hbm.at[pl.ds(base, bpw)])

    return gk(flat_idx, cls3, mem2)


def kernel(enc_outputs_class, enc_outputs_coord_logits_plus_anchors,
           output_memory, sources_last_element):
    cm3, clspad = _row_max(enc_outputs_class)
    cm16 = cm3.reshape(B, 1, N)
    coordt = jnp.transpose(enc_outputs_coord_logits_plus_anchors, (0, 2, 1))
    idxp, refp, bbox = _topk(cm16, coordt)
    flat_idx = idxp.reshape(B * KPAD)
    logits_p, target_p = _sc_gather(
        flat_idx,
        clspad.reshape(B * N, 128),
        output_memory.reshape(B * N, DMEM))
    init_reference_points = jnp.transpose(refp, (0, 2, 1))[:, :K]
    enc_topk_bboxes = jnp.transpose(bbox, (0, 2, 1))[:, :K]
    enc_topk_logits = logits_p.reshape(B, KPAD, 128)[:, :K, :NCLS]
    target = target_p.reshape(B, KPAD, DMEM)[:, :K]
    return (init_reference_points, target, enc_topk_logits, enc_topk_bboxes)


# speedup vs baseline: 1.1377x; 1.1377x over previous
import functools

import jax
import jax.numpy as jnp
from jax import lax
from jax.experimental import pallas as pl
from jax.experimental.pallas import tpu as pltpu
from jax.experimental.pallas import tpu_sc as plsc

K = 300
KPAD = 320
CAND = 384
CHUNK = 128
B, N, NCLS, DMEM = 16, 8192, 80, 256
INT_MIN32 = -2147483648


def _rowmax_body(cls_ref, out_ref, pad_ref):
    x = cls_ref[0]
    out_ref[0, 0, :] = jnp.max(x, axis=-1)
    pad_ref[0] = jnp.concatenate(
        [x, jnp.zeros((N // 4, 128 - NCLS), jnp.float32)], axis=-1)


def _row_max(cls):
    return pl.pallas_call(
        _rowmax_body,
        grid=(B * 4,),
        in_specs=[pl.BlockSpec((1, N // 4, NCLS), lambda i: (i // 4, i % 4, 0))],
        out_specs=[pl.BlockSpec((1, 1, N // 4), lambda i: (i, 0, 0)),
                   pl.BlockSpec((1, N // 4, 128), lambda i: (i // 4, i % 4, 0))],
        out_shape=[jax.ShapeDtypeStruct((B * 4, 1, N // 4), jnp.float32),
                   jax.ShapeDtypeStruct((B, N, 128), jnp.float32)],
    )(cls)


def _radix_body(cm_ref, ts_ref):
    x = cm_ref[...]
    kb = lax.bitcast_convert_type(x, jnp.int32)
    m = jnp.where(kb < 0, kb ^ jnp.int32(0x7FFFFFFF), kb)
    tb = jnp.zeros((B, 1), jnp.int32)
    for bit in range(31, -1, -1):
        bv = jnp.int32(INT_MIN32) if bit == 31 else jnp.int32(1 << bit)
        cand = tb | bv
        ts = cand ^ jnp.int32(INT_MIN32)
        cnt = jnp.sum((m >= ts).astype(jnp.int32), axis=1, keepdims=True)
        tb = jnp.where(cnt >= K, cand, tb)
    ts_ref[...] = tb ^ jnp.int32(INT_MIN32)


def _radix(cm):
    return pl.pallas_call(
        _radix_body,
        out_shape=jax.ShapeDtypeStruct((B, 1), jnp.int32),
    )(cm)


def _topk_body(cm_ref, coordt_ref, ts_ref, idx_ref, refp_ref, bbox_ref):
    b = pl.program_id(0)
    x = cm_ref[0]
    kb = lax.bitcast_convert_type(x, jnp.int32)
    m = jnp.where(kb < 0, kb ^ jnp.int32(0x7FFFFFFF), kb)
    m2 = m.reshape(64, 128)
    ts = ts_ref[0]
    flag2 = m2 >= ts
    fl2 = flag2.astype(jnp.float32)
    count_ge = jnp.sum(fl2)

    lt128 = (lax.broadcasted_iota(jnp.int32, (128, 128), 0)
             < lax.broadcasted_iota(jnp.int32, (128, 128), 1)).astype(jnp.float32)
    ex = jnp.dot(fl2, lt128, preferred_element_type=jnp.float32)
    rows = jnp.sum(fl2, axis=1, keepdims=True)
    ltt64 = (lax.broadcasted_iota(jnp.int32, (64, 64), 1)
             < lax.broadcasted_iota(jnp.int32, (64, 64), 0)).astype(jnp.float32)
    rowex = jnp.dot(ltt64, rows, preferred_element_type=jnp.float32)
    pos2 = rowex + ex
    posm = jnp.where(flag2, pos2, -1.0).reshape(1, N)

    gidxf = (lax.broadcasted_iota(jnp.int32, (1, N), 1)
             + b * N).astype(jnp.float32)
    vt = jnp.concatenate([x, gidxf, coordt_ref[0]], axis=0)

    jio = lax.broadcasted_iota(jnp.int32, (CAND, N // 8), 0).astype(jnp.float32)
    acc = jnp.zeros((CAND, 6), jnp.float32)
    for c in range(8):
        sl = slice(c * (N // 8), (c + 1) * (N // 8))
        p = (jio == posm[:, sl]).astype(jnp.float32)
        acc = acc + lax.dot_general(p, vt[:, sl], (((1,), (1,)), ((), ())),
                                    precision=lax.Precision.HIGHEST,
                                    preferred_element_type=jnp.float32)

    slotc = lax.broadcasted_iota(jnp.int32, (CAND, 1), 0).astype(jnp.float32)
    valid = slotc < count_ge
    cval = jnp.where(valid, acc[:, 0:1], -jnp.inf)
    cidx = jnp.where(valid, acc[:, 1:2], jnp.float32(2 * B * N))

    eye = (lax.broadcasted_iota(jnp.int32, (CAND, CAND), 0)
           == lax.broadcasted_iota(jnp.int32, (CAND, CAND), 1)).astype(jnp.float32)
    avr = lax.dot_general(acc[:, 0:1], eye, (((0,), (0,)), ((), ())),
                          precision=lax.Precision.HIGHEST,
                          preferred_element_type=jnp.float32)
    air = lax.dot_general(acc[:, 1:2], eye, (((0,), (0,)), ((), ())),
                          precision=lax.Precision.HIGHEST,
                          preferred_element_type=jnp.float32)
    validr = (lax.broadcasted_iota(jnp.int32, (1, CAND), 1)
              .astype(jnp.float32) < count_ge)
    cvalr = jnp.where(validr, avr, -jnp.inf)
    cidxr = jnp.where(validr, air, jnp.float32(2 * B * N))

    gt = (cvalr > cval) | ((cvalr == cval) & (cidxr < cidx))
    cnt2 = jnp.sum((gt & validr).astype(jnp.float32), axis=1, keepdims=True)
    rank = jnp.where(valid, cnt2, jnp.float32(CAND))

    oh = (rank == lax.broadcasted_iota(jnp.int32, (CAND, KPAD), 1)
          .astype(jnp.float32)).astype(jnp.float32)
    v2 = jnp.concatenate([cidx, acc[:, 2:6]], axis=1)
    out5 = lax.dot_general(v2, oh, (((0,), (0,)), ((), ())),
                           precision=lax.Precision.HIGHEST,
                           preferred_element_type=jnp.float32)
    idx_ref[0] = out5[0:1, :].astype(jnp.int32)
    rp = out5[1:5, :]
    refp_ref[0] = rp
    bbox_ref[0] = jax.nn.sigmoid(rp)


def _topk(cm16, coordt, ts16):
    return pl.pallas_call(
        _topk_body,
        grid=(B,),
        in_specs=[pl.BlockSpec((1, 1, N), lambda i: (i, 0, 0)),
                  pl.BlockSpec((1, 4, N), lambda i: (i, 0, 0)),
                  pl.BlockSpec((1, 1, 1), lambda i: (i, 0, 0))],
        out_specs=[pl.BlockSpec((1, 1, KPAD), lambda i: (i, 0, 0)),
                   pl.BlockSpec((1, 4, KPAD), lambda i: (i, 0, 0)),
                   pl.BlockSpec((1, 4, KPAD), lambda i: (i, 0, 0))],
        out_shape=[jax.ShapeDtypeStruct((B, 1, KPAD), jnp.int32),
                   jax.ShapeDtypeStruct((B, 4, KPAD), jnp.float32),
                   jax.ShapeDtypeStruct((B, 4, KPAD), jnp.float32)],
    )(cm16, coordt, ts16)


def _sc_gather(flat_idx, cls3, mem2):
    info = plsc.get_sparse_core_info()
    nw = info.num_cores * info.num_subcores
    lanes = info.num_lanes
    tot = B * KPAD
    bpw = tot // nw
    nch = -(-bpw // 128)
    cw = bpw // nch
    mesh = plsc.VectorSubcoreMesh(core_axis_name="c", subcore_axis_name="s")

    @functools.partial(
        pl.kernel, mesh=mesh,
        out_type=[jax.ShapeDtypeStruct((tot, 128), jnp.float32),
                  jax.ShapeDtypeStruct((tot, DMEM), jnp.float32)],
        scratch_types=[pltpu.VMEM((bpw,), jnp.int32),
                       pltpu.VMEM((bpw, 128), jnp.float32),
                       pltpu.VMEM((bpw, DMEM), jnp.float32),
                       pltpu.SemaphoreType.DMA,
                       pltpu.SemaphoreType.DMA],
    )
    def gk(idx_hbm, cls_hbm, mem_hbm, ocls_hbm, omem_hbm,
           idx_v, crows, mrows, sem_c, sem_m):
        wid = lax.axis_index("s") * info.num_cores + lax.axis_index("c")
        base = wid * bpw
        pltpu.sync_copy(idx_hbm.at[pl.ds(base, bpw)], idx_v)
        cps = []
        for c in range(nch):
            sl = pl.ds(c * cw, cw)
            cps.append(pltpu.async_copy(cls_hbm.at[idx_v.at[sl]], crows.at[sl], sem_c))
            cps.append(pltpu.async_copy(mem_hbm.at[idx_v.at[sl]], mrows.at[sl], sem_m))
        for cp in cps:
            cp.wait()
        pltpu.sync_copy(crows, ocls_hbm.at[pl.ds(base, bpw)])
        pltpu.sync_copy(mrows, omem_hbm.at[pl.ds(base, bpw)])

    return gk(flat_idx, cls3, mem2)


def kernel(enc_outputs_class, enc_outputs_coord_logits_plus_anchors,
           output_memory, sources_last_element):
    cm3, clspad = _row_max(enc_outputs_class)
    cm16 = cm3.reshape(B, 1, N)
    ts16 = _radix(cm3.reshape(B, N)).reshape(B, 1, 1)
    coordt = jnp.transpose(enc_outputs_coord_logits_plus_anchors, (0, 2, 1))
    idxp, refp, bbox = _topk(cm16, coordt, ts16)
    flat_idx = idxp.reshape(B * KPAD)
    logits_p, target_p = _sc_gather(
        flat_idx,
        clspad.reshape(B * N, 128),
        output_memory.reshape(B * N, DMEM))
    init_reference_points = jnp.transpose(refp, (0, 2, 1))[:, :K]
    enc_topk_bboxes = jnp.transpose(bbox, (0, 2, 1))[:, :K]
    enc_topk_logits = logits_p.reshape(B, KPAD, 128)[:, :K, :NCLS]
    target = target_p.reshape(B, KPAD, DMEM)[:, :K]
    return (init_reference_points, target, enc_topk_logits, enc_topk_bboxes)


# speedup vs baseline: 1.2404x; 1.0903x over previous
import functools

import jax
import jax.numpy as jnp
from jax import lax
from jax.experimental import pallas as pl
from jax.experimental.pallas import tpu as pltpu
from jax.experimental.pallas import tpu_sc as plsc

K = 300
KPAD = 320
CAND = 320
CHUNK = 128
B, N, NCLS, DMEM = 16, 8192, 80, 256
INT_MIN32 = -2147483648


def _rowmax_body(cls_ref, out_ref, pad_ref):
    x = cls_ref[0]
    out_ref[0, 0, :] = jnp.max(x, axis=-1)
    pad_ref[0] = jnp.concatenate(
        [x, jnp.zeros((N // 4, 128 - NCLS), jnp.float32)], axis=-1)


def _row_max(cls):
    return pl.pallas_call(
        _rowmax_body,
        grid=(B * 4,),
        in_specs=[pl.BlockSpec((1, N // 4, NCLS), lambda i: (i // 4, i % 4, 0))],
        out_specs=[pl.BlockSpec((1, 1, N // 4), lambda i: (i, 0, 0)),
                   pl.BlockSpec((1, N // 4, 128), lambda i: (i // 4, i % 4, 0))],
        out_shape=[jax.ShapeDtypeStruct((B * 4, 1, N // 4), jnp.float32),
                   jax.ShapeDtypeStruct((B, N, 128), jnp.float32)],
    )(cls)


def _radix_body(cm_ref, ts_ref):
    x = cm_ref[...]
    kb = lax.bitcast_convert_type(x, jnp.int32)
    m = jnp.where(kb < 0, kb ^ jnp.int32(0x7FFFFFFF), kb)
    tb = jnp.zeros((B, 1), jnp.int32)
    mn = jnp.int32(INT_MIN32)

    def _count(t):
        return jnp.sum((m >= (t ^ mn)).astype(jnp.int32), axis=1, keepdims=True)

    for hi in range(31, 0, -2):
        b1 = mn if hi == 31 else jnp.int32(1 << hi)
        b0 = jnp.int32(1 << (hi - 1))
        c10, c01, c11 = tb | b1, tb | b0, tb | b1 | b0
        n10, n01, n11 = _count(c10), _count(c01), _count(c11)
        tb = jnp.where(n10 >= K,
                       jnp.where(n11 >= K, c11, c10),
                       jnp.where(n01 >= K, c01, tb))
    ts_ref[...] = tb ^ mn


def _radix(cm):
    return pl.pallas_call(
        _radix_body,
        out_shape=jax.ShapeDtypeStruct((B, 1), jnp.int32),
    )(cm)


def _topk_body(cm_ref, coordt_ref, ts_ref, idx_ref, refp_ref, bbox_ref):
    b = pl.program_id(0)
    x = cm_ref[0]
    kb = lax.bitcast_convert_type(x, jnp.int32)
    m = jnp.where(kb < 0, kb ^ jnp.int32(0x7FFFFFFF), kb)
    m2 = m.reshape(64, 128)
    ts = ts_ref[0]
    flag2 = m2 >= ts
    fl2 = flag2.astype(jnp.float32)
    count_ge = jnp.sum(fl2)

    lt128 = (lax.broadcasted_iota(jnp.int32, (128, 128), 0)
             < lax.broadcasted_iota(jnp.int32, (128, 128), 1)).astype(jnp.float32)
    ex = jnp.dot(fl2, lt128, preferred_element_type=jnp.float32)
    rows = jnp.sum(fl2, axis=1, keepdims=True)
    ltt64 = (lax.broadcasted_iota(jnp.int32, (64, 64), 1)
             < lax.broadcasted_iota(jnp.int32, (64, 64), 0)).astype(jnp.float32)
    rowex = jnp.dot(ltt64, rows, preferred_element_type=jnp.float32)
    pos2 = rowex + ex
    posm = jnp.where(flag2, pos2, -1.0).reshape(1, N)

    gidxf = (lax.broadcasted_iota(jnp.int32, (1, N), 1)
             + b * N).astype(jnp.float32)
    vt = jnp.concatenate([x, gidxf, coordt_ref[0]], axis=0)

    jio = lax.broadcasted_iota(jnp.int32, (CAND, N // 8), 0).astype(jnp.float32)
    acc = jnp.zeros((CAND, 6), jnp.float32)
    for c in range(8):
        sl = slice(c * (N // 8), (c + 1) * (N // 8))
        p = (jio == posm[:, sl]).astype(jnp.float32)
        acc = acc + lax.dot_general(p, vt[:, sl], (((1,), (1,)), ((), ())),
                                    precision=lax.Precision.HIGHEST,
                                    preferred_element_type=jnp.float32)

    slotc = lax.broadcasted_iota(jnp.int32, (CAND, 1), 0).astype(jnp.float32)
    valid = slotc < count_ge
    cval = jnp.where(valid, acc[:, 0:1], -jnp.inf)
    cidx = jnp.where(valid, acc[:, 1:2], jnp.float32(2 * B * N))

    eye = (lax.broadcasted_iota(jnp.int32, (CAND, CAND), 0)
           == lax.broadcasted_iota(jnp.int32, (CAND, CAND), 1)).astype(jnp.float32)
    avr = lax.dot_general(acc[:, 0:1], eye, (((0,), (0,)), ((), ())),
                          precision=lax.Precision.HIGHEST,
                          preferred_element_type=jnp.float32)
    air = lax.dot_general(acc[:, 1:2], eye, (((0,), (0,)), ((), ())),
                          precision=lax.Precision.HIGHEST,
                          preferred_element_type=jnp.float32)
    validr = (lax.broadcasted_iota(jnp.int32, (1, CAND), 1)
              .astype(jnp.float32) < count_ge)
    cvalr = jnp.where(validr, avr, -jnp.inf)
    cidxr = jnp.where(validr, air, jnp.float32(2 * B * N))

    gt = (cvalr > cval) | ((cvalr == cval) & (cidxr < cidx))
    cnt2 = jnp.sum((gt & validr).astype(jnp.float32), axis=1, keepdims=True)
    rank = jnp.where(valid, cnt2, jnp.float32(CAND))

    oh = (rank == lax.broadcasted_iota(jnp.int32, (CAND, KPAD), 1)
          .astype(jnp.float32)).astype(jnp.float32)
    v2 = jnp.concatenate([cidx, acc[:, 2:6]], axis=1)
    out5 = lax.dot_general(v2, oh, (((0,), (0,)), ((), ())),
                           precision=lax.Precision.HIGHEST,
                           preferred_element_type=jnp.float32)
    idx_ref[0] = out5[0:1, :].astype(jnp.int32)
    rp = out5[1:5, :]
    refp_ref[0] = rp
    bbox_ref[0] = jax.nn.sigmoid(rp)


def _topk(cm16, coordt, ts16):
    return pl.pallas_call(
        _topk_body,
        grid=(B,),
        in_specs=[pl.BlockSpec((1, 1, N), lambda i: (i, 0, 0)),
                  pl.BlockSpec((1, 4, N), lambda i: (i, 0, 0)),
                  pl.BlockSpec((1, 1, 1), lambda i: (i, 0, 0))],
        out_specs=[pl.BlockSpec((1, 1, KPAD), lambda i: (i, 0, 0)),
                   pl.BlockSpec((1, 4, KPAD), lambda i: (i, 0, 0)),
                   pl.BlockSpec((1, 4, KPAD), lambda i: (i, 0, 0))],
        out_shape=[jax.ShapeDtypeStruct((B, 1, KPAD), jnp.int32),
                   jax.ShapeDtypeStruct((B, 4, KPAD), jnp.float32),
                   jax.ShapeDtypeStruct((B, 4, KPAD), jnp.float32)],
    )(cm16, coordt, ts16)


def _sc_gather(flat_idx, cls3, mem2):
    info = plsc.get_sparse_core_info()
    nw = info.num_cores * info.num_subcores
    lanes = info.num_lanes
    tot = B * KPAD
    bpw = tot // nw
    nch = -(-bpw // 128)
    cw = bpw // nch
    mesh = plsc.VectorSubcoreMesh(core_axis_name="c", subcore_axis_name="s")

    @functools.partial(
        pl.kernel, mesh=mesh,
        out_type=[jax.ShapeDtypeStruct((tot, 128), jnp.float32),
                  jax.ShapeDtypeStruct((tot, DMEM), jnp.float32)],
        scratch_types=[pltpu.VMEM((bpw,), jnp.int32),
                       pltpu.VMEM((bpw, 128), jnp.float32),
                       pltpu.VMEM((bpw, DMEM), jnp.float32),
                       pltpu.SemaphoreType.DMA,
                       pltpu.SemaphoreType.DMA],
    )
    def gk(idx_hbm, cls_hbm, mem_hbm, ocls_hbm, omem_hbm,
           idx_v, crows, mrows, sem_c, sem_m):
        wid = lax.axis_index("s") * info.num_cores + lax.axis_index("c")
        base = wid * bpw
        pltpu.sync_copy(idx_hbm.at[pl.ds(base, bpw)], idx_v)
        cps = []
        for c in range(nch):
            sl = pl.ds(c * cw, cw)
            cps.append(pltpu.async_copy(cls_hbm.at[idx_v.at[sl]], crows.at[sl], sem_c))
            cps.append(pltpu.async_copy(mem_hbm.at[idx_v.at[sl]], mrows.at[sl], sem_m))
        for cp in cps:
            cp.wait()
        pltpu.sync_copy(crows, ocls_hbm.at[pl.ds(base, bpw)])
        pltpu.sync_copy(mrows, omem_hbm.at[pl.ds(base, bpw)])

    return gk(flat_idx, cls3, mem2)


def kernel(enc_outputs_class, enc_outputs_coord_logits_plus_anchors,
           output_memory, sources_last_element):
    cm3, clspad = _row_max(enc_outputs_class)
    cm16 = cm3.reshape(B, 1, N)
    ts16 = _radix(cm3.reshape(B, N)).reshape(B, 1, 1)
    coordt = jnp.transpose(enc_outputs_coord_logits_plus_anchors, (0, 2, 1))
    idxp, refp, bbox = _topk(cm16, coordt, ts16)
    flat_idx = idxp.reshape(B * KPAD)
    logits_p, target_p = _sc_gather(
        flat_idx,
        clspad.reshape(B * N, 128),
        output_memory.reshape(B * N, DMEM))
    init_reference_points = jnp.transpose(refp, (0, 2, 1))[:, :K]
    enc_topk_bboxes = jnp.transpose(bbox, (0, 2, 1))[:, :K]
    enc_topk_logits = logits_p.reshape(B, KPAD, 128)[:, :K, :NCLS]
    target = target_p.reshape(B, KPAD, DMEM)[:, :K]
    return (init_reference_points, target, enc_topk_logits, enc_topk_bboxes)
